# baseline (speedup 1.0000x reference)
import jax
import jax.numpy as jnp
from jax import lax
from jax.experimental import pallas as pl
from jax.experimental.pallas import tpu as pltpu
from jax.experimental.pallas import tpu_sc as plsc

VOCAB = 100000
HIDDEN = 768
BATCH = 4
SEQ = 8192

NC, NS, L = 2, 16, 16
NW = NC * NS
SEQ_PER_W = SEQ // NW
CS = 8
NCHUNK = SEQ_PER_W // CS
NB = 4
HV = HIDDEN // L


def _embed_body(ids_hbm, tok_hbm, pos_hbm, out_hbm,
                idx_v, tok_v, pos_v, gsem, ssem, isem):
    wid = lax.axis_index("s") * NC + lax.axis_index("c")
    base = wid * SEQ_PER_W

    def fire_pos(c, k):
        off = c * CS
        pltpu.async_copy(pos_hbm.at[pl.ds(base + off, CS)],
                         pos_v.at[k], gsem.at[k])

    def fire_tok(c, k):
        off = c * CS
        for b in range(BATCH):
            pltpu.async_copy(tok_hbm.at[idx_v.at[b, pl.ds(off, CS)]],
                             tok_v.at[k, b], gsem.at[k])

    def fire_gather(c, k):
        fire_pos(c, k)
        fire_tok(c, k)

    def wait_gather(k):
        pltpu.make_async_copy(pos_hbm.at[pl.ds(0, CS)],
                              pos_v.at[k], gsem.at[k]).wait()
        for b in range(BATCH):
            pltpu.make_async_copy(tok_hbm.at[pl.ds(0, CS)],
                                  tok_v.at[k, b], gsem.at[k]).wait()

    def fire_store(c, k):
        off = base + c * CS
        for b in range(BATCH):
            pltpu.async_copy(tok_v.at[k, b],
                             out_hbm.at[b, pl.ds(off, CS)], ssem.at[k])

    def wait_store(k):
        for b in range(BATCH):
            pltpu.make_async_copy(tok_v.at[k, b],
                                  out_hbm.at[b, pl.ds(0, CS)], ssem.at[k]).wait()

    def add_slot(k):
        def rb_body(j, rc):
            b = j // CS
            r = j % CS
            for h in range(HV):
                sl = pl.ds(h * L, L)
                plsc.addupdate(tok_v.at[k, b, r, sl], pos_v[k, r, sl])
            return rc
        lax.fori_loop(0, BATCH * CS, rb_body, 0)

    idx_cps = [pltpu.async_copy(ids_hbm.at[b, pl.ds(base, SEQ_PER_W)],
                                idx_v.at[b], isem) for b in range(BATCH)]
    fire_pos(0, 0)
    fire_pos(1, 1)
    fire_pos(2, 2)
    for cp in idx_cps:
        cp.wait()
    fire_tok(0, 0)
    fire_tok(1, 1)
    fire_tok(2, 2)

    def outer(t, carry):
        for ks in range(NB):
            c = t * NB + ks
            k2 = (ks + 3) % NB

            @pl.when(jnp.logical_and(c >= 1, c + 3 < NCHUNK))
            def _():
                wait_store(k2)

            @pl.when(c + 3 < NCHUNK)
            def _():
                fire_gather(c + 3, k2)

            wait_gather(ks)
            add_slot(ks)
            fire_store(c, ks)
        return carry

    lax.fori_loop(0, NCHUNK // NB, outer, 0)

    for kk in range(NB):
        wait_store((NCHUNK - 4 + kk) % NB)


def kernel(input_ids, tok_embed, pos_embed, past_seq_len=0):
    del past_seq_len
    ids32 = input_ids.astype(jnp.int32)

    mesh = plsc.VectorSubcoreMesh(core_axis_name="c", subcore_axis_name="s")
    run = pl.kernel(
        _embed_body,
        out_type=jax.ShapeDtypeStruct((BATCH, SEQ, HIDDEN), jnp.float32),
        mesh=mesh,
        scratch_types=[
            pltpu.VMEM((BATCH, SEQ_PER_W), jnp.int32),
            pltpu.VMEM((NB, BATCH, CS, HIDDEN), jnp.float32),
            pltpu.VMEM((NB, CS, HIDDEN), jnp.float32),
            pltpu.SemaphoreType.DMA((NB,)),
            pltpu.SemaphoreType.DMA((NB,)),
            pltpu.SemaphoreType.DMA,
        ],
    )
    return run(ids32, tok_embed, pos_embed)


# speedup vs baseline: 2.0619x; 2.0619x over previous
import jax
import jax.numpy as jnp
from jax import lax
from jax.experimental import pallas as pl
from jax.experimental.pallas import tpu as pltpu
from jax.experimental.pallas import tpu_sc as plsc

VOCAB = 100000
HIDDEN = 768
BATCH = 4
SEQ = 8192

NC, NS, L = 2, 16, 16
NW = NC * NS
SEQ_PER_W = SEQ // NW
CS = 8
NCHUNK = SEQ_PER_W // CS
NB = 4
HV = HIDDEN // L


def _embed_body(ids_hbm, tok_hbm, pos_hbm, out_hbm,
                idx_v, tok_v, pos_v, gsem, ssem, isem):
    wid = lax.axis_index("s") * NC + lax.axis_index("c")
    base = wid * SEQ_PER_W

    def fire_pos(c, k):
        off = c * CS
        pltpu.async_copy(pos_hbm.at[pl.ds(base + off, CS)],
                         pos_v.at[k], gsem.at[k])

    def fire_tok(c, k):
        off = c * CS
        for b in range(BATCH):
            pltpu.async_copy(tok_hbm.at[idx_v.at[b, pl.ds(off, CS)]],
                             tok_v.at[k, b], gsem.at[k])

    def fire_gather(c, k):
        fire_pos(c, k)
        fire_tok(c, k)

    def wait_gather(k):
        pltpu.make_async_copy(pos_hbm.at[pl.ds(0, CS)],
                              pos_v.at[k], gsem.at[k]).wait()
        for b in range(BATCH):
            pltpu.make_async_copy(tok_hbm.at[pl.ds(0, CS)],
                                  tok_v.at[k, b], gsem.at[k]).wait()

    def fire_store_b(c, k, b):
        off = base + c * CS
        pltpu.async_copy(tok_v.at[k, b],
                         out_hbm.at[b, pl.ds(off, CS)], ssem.at[k])

    def wait_store(k):
        for b in range(BATCH):
            pltpu.make_async_copy(tok_v.at[k, b],
                                  out_hbm.at[b, pl.ds(0, CS)], ssem.at[k]).wait()

    def add_batch(k, b):
        def row_body(r, rc):
            for h in range(HV):
                sl = pl.ds(h * L, L)
                plsc.addupdate(tok_v.at[k, b, r, sl], pos_v[k, r, sl])
            return rc
        lax.fori_loop(0, CS, row_body, 0)

    idx_cps = [pltpu.async_copy(ids_hbm.at[b, pl.ds(base, SEQ_PER_W)],
                                idx_v.at[b], isem) for b in range(BATCH)]
    fire_pos(0, 0)
    fire_pos(1, 1)
    fire_pos(2, 2)
    for cp in idx_cps:
        cp.wait()
    fire_tok(0, 0)
    fire_tok(1, 1)
    fire_tok(2, 2)

    def outer(t, carry):
        for ks in range(NB):
            c = t * NB + ks
            k2 = (ks + 3) % NB

            @pl.when(jnp.logical_and(c >= 1, c + 3 < NCHUNK))
            def _():
                wait_store(k2)

            @pl.when(c + 3 < NCHUNK)
            def _():
                fire_gather(c + 3, k2)

            wait_gather(ks)
            for b in range(BATCH):
                add_batch(ks, b)
                fire_store_b(c, ks, b)
        return carry

    lax.fori_loop(0, NCHUNK // NB, outer, 0)

    for kk in range(NB):
        wait_store((NCHUNK - 4 + kk) % NB)


def kernel(input_ids, tok_embed, pos_embed, past_seq_len=0):
    del past_seq_len
    ids32 = input_ids.astype(jnp.int32)

    mesh = plsc.VectorSubcoreMesh(core_axis_name="c", subcore_axis_name="s")
    run = pl.kernel(
        _embed_body,
        out_type=jax.ShapeDtypeStruct((BATCH, SEQ, HIDDEN), jnp.float32),
        mesh=mesh,
        scratch_types=[
            pltpu.VMEM((BATCH, SEQ_PER_W), jnp.int32),
            pltpu.VMEM((NB, BATCH, CS, HIDDEN), jnp.float32),
            pltpu.VMEM((NB, CS, HIDDEN), jnp.float32),
            pltpu.SemaphoreType.DMA((NB,)),
            pltpu.SemaphoreType.DMA((NB,)),
            pltpu.SemaphoreType.DMA,
        ],
    )
    return run(ids32, tok_embed, pos_embed)


# speedup vs baseline: 2.1775x; 1.0561x over previous
import jax
import jax.numpy as jnp
from jax import lax
from jax.experimental import pallas as pl
from jax.experimental.pallas import tpu as pltpu
from jax.experimental.pallas import tpu_sc as plsc

VOCAB = 100000
HIDDEN = 768
BATCH = 4
SEQ = 8192

NC, NS, L = 2, 16, 16
NW = NC * NS
SEQ_PER_W = SEQ // NW
CS = 8
NCHUNK = SEQ_PER_W // CS
NB = 4
HV = HIDDEN // L


def _embed_body(ids_hbm, tok_hbm, pos_hbm, out_hbm,
                idx_v, tok_v, pos_v, gsem, ssem, isem):
    wid = lax.axis_index("s") * NC + lax.axis_index("c")
    base = wid * SEQ_PER_W

    def fire_pos(c, k):
        off = c * CS
        pltpu.async_copy(pos_hbm.at[pl.ds(base + off, CS)],
                         pos_v.at[k], gsem.at[k])

    def fire_tok(c, k):
        off = c * CS
        for b in range(BATCH):
            pltpu.async_copy(tok_hbm.at[idx_v.at[b, pl.ds(off, CS)]],
                             tok_v.at[k, b], gsem.at[k])

    def fire_gather(c, k):
        fire_pos(c, k)
        fire_tok(c, k)

    def wait_gather(k):
        pltpu.make_async_copy(pos_hbm.at[pl.ds(0, CS)],
                              pos_v.at[k], gsem.at[k]).wait()
        for b in range(BATCH):
            pltpu.make_async_copy(tok_hbm.at[pl.ds(0, CS)],
                                  tok_v.at[k, b], gsem.at[k]).wait()

    def fire_store(c, k):
        off = base + c * CS
        for b in range(BATCH):
            pltpu.async_copy(tok_v.at[k, b],
                             out_hbm.at[b, pl.ds(off, CS)], ssem.at[k])

    def wait_store(k):
        for b in range(BATCH):
            pltpu.make_async_copy(tok_v.at[k, b],
                                  out_hbm.at[b, pl.ds(0, CS)], ssem.at[k]).wait()

    def add_slot(k):
        def row_body(r, rc):
            for h in range(HV):
                sl = pl.ds(h * L, L)
                p = pos_v[k, r, sl]
                for b in range(BATCH):
                    plsc.addupdate(tok_v.at[k, b, r, sl], p)
            return rc
        lax.fori_loop(0, CS, row_body, 0)

    idx_cps = [pltpu.async_copy(ids_hbm.at[b, pl.ds(base, SEQ_PER_W)],
                                idx_v.at[b], isem) for b in range(BATCH)]
    fire_pos(0, 0)
    fire_pos(1, 1)
    fire_pos(2, 2)
    for cp in idx_cps:
        cp.wait()
    fire_tok(0, 0)
    fire_tok(1, 1)
    fire_tok(2, 2)

    def outer(t, carry):
        for ks in range(NB):
            c = t * NB + ks
            k2 = (ks + 3) % NB

            @pl.when(jnp.logical_and(c >= 1, c + 3 < NCHUNK))
            def _():
                wait_store(k2)

            @pl.when(c + 3 < NCHUNK)
            def _():
                fire_gather(c + 3, k2)

            wait_gather(ks)
            add_slot(ks)
            fire_store(c, ks)
        return carry

    lax.fori_loop(0, NCHUNK // NB, outer, 0)

    for kk in range(NB):
        wait_store((NCHUNK - 4 + kk) % NB)


def kernel(input_ids, tok_embed, pos_embed, past_seq_len=0):
    del past_seq_len
    ids32 = input_ids.astype(jnp.int32)

    mesh = plsc.VectorSubcoreMesh(core_axis_name="c", subcore_axis_name="s")
    run = pl.kernel(
        _embed_body,
        out_type=jax.ShapeDtypeStruct((BATCH, SEQ, HIDDEN), jnp.float32),
        mesh=mesh,
        scratch_types=[
            pltpu.VMEM((BATCH, SEQ_PER_W), jnp.int32),
            pltpu.VMEM((NB, BATCH, CS, HIDDEN), jnp.float32),
            pltpu.VMEM((NB, CS, HIDDEN), jnp.float32),
            pltpu.SemaphoreType.DMA((NB,)),
            pltpu.SemaphoreType.DMA((NB,)),
            pltpu.SemaphoreType.DMA,
        ],
    )
    return run(ids32, tok_embed, pos_embed)


# speedup vs baseline: 2.1863x; 1.0040x over previous
import jax
import jax.numpy as jnp
from jax import lax
from jax.experimental import pallas as pl
from jax.experimental.pallas import tpu as pltpu
from jax.experimental.pallas import tpu_sc as plsc

VOCAB = 100000
HIDDEN = 768
BATCH = 4
SEQ = 8192

NC, NS, L = 2, 16, 16
NW = NC * NS
SEQ_PER_W = SEQ // NW
CS = 8
NCHUNK = SEQ_PER_W // CS
NB = 4
HV = HIDDEN // L


def _embed_body(ids_hbm, tok_hbm, pos_hbm, out_hbm,
                idx_v, tok_v, pos_v, gsem, ssem, isem):
    wid = lax.axis_index("s") * NC + lax.axis_index("c")
    base = wid * SEQ_PER_W

    def fire_pos(c, k):
        off = c * CS
        pltpu.async_copy(pos_hbm.at[pl.ds(base + off, CS)],
                         pos_v.at[k], gsem.at[k])

    def fire_tok(c, k):
        off = c * CS
        for b in range(BATCH):
            pltpu.async_copy(tok_hbm.at[idx_v.at[b, pl.ds(off, CS)]],
                             tok_v.at[k, pl.ds(b * CS, CS)], gsem.at[k])

    def fire_gather(c, k):
        fire_pos(c, k)
        fire_tok(c, k)

    def wait_gather(k):
        pltpu.make_async_copy(pos_hbm.at[pl.ds(0, CS)],
                              pos_v.at[k], gsem.at[k]).wait()
        pltpu.make_async_copy(tok_hbm.at[pl.ds(0, BATCH * CS)],
                              tok_v.at[k], gsem.at[k]).wait()

    def fire_store(c, k):
        off = base + c * CS
        for b in range(BATCH):
            pltpu.async_copy(tok_v.at[k, pl.ds(b * CS, CS)],
                             out_hbm.at[b, pl.ds(off, CS)], ssem.at[k])

    def wait_store(k):
        pltpu.make_async_copy(tok_v.at[k],
                              out_hbm.at[0, pl.ds(0, BATCH * CS)],
                              ssem.at[k]).wait()

    def add_slot(k):
        def row_body(r, rc):
            for h in range(HV):
                sl = pl.ds(h * L, L)
                p = pos_v[k, r, sl]
                for b in range(BATCH):
                    plsc.addupdate(tok_v.at[k, b * CS + r, sl], p)
            return rc
        lax.fori_loop(0, CS, row_body, 0)

    idx_cps = [pltpu.async_copy(ids_hbm.at[b, pl.ds(base, SEQ_PER_W)],
                                idx_v.at[b], isem) for b in range(BATCH)]
    fire_pos(0, 0)
    fire_pos(1, 1)
    fire_pos(2, 2)
    for cp in idx_cps:
        cp.wait()
    fire_tok(0, 0)
    fire_tok(1, 1)
    fire_tok(2, 2)

    def outer(t, carry):
        for ks in range(NB):
            c = t * NB + ks
            k2 = (ks + 3) % NB

            @pl.when(jnp.logical_and(c >= 1, c + 3 < NCHUNK))
            def _():
                wait_store(k2)

            @pl.when(c + 3 < NCHUNK)
            def _():
                fire_gather(c + 3, k2)

            wait_gather(ks)
            add_slot(ks)
            fire_store(c, ks)
        return carry

    lax.fori_loop(0, NCHUNK // NB, outer, 0)

    for kk in range(NB):
        wait_store((NCHUNK - 4 + kk) % NB)


def kernel(input_ids, tok_embed, pos_embed, past_seq_len=0):
    del past_seq_len
    ids32 = input_ids.astype(jnp.int32)

    mesh = plsc.VectorSubcoreMesh(core_axis_name="c", subcore_axis_name="s")
    run = pl.kernel(
        _embed_body,
        out_type=jax.ShapeDtypeStruct((BATCH, SEQ, HIDDEN), jnp.float32),
        mesh=mesh,
        scratch_types=[
            pltpu.VMEM((BATCH, SEQ_PER_W), jnp.int32),
            pltpu.VMEM((NB, BATCH * CS, HIDDEN), jnp.float32),
            pltpu.VMEM((NB, CS, HIDDEN), jnp.float32),
            pltpu.SemaphoreType.DMA((NB,)),
            pltpu.SemaphoreType.DMA((NB,)),
            pltpu.SemaphoreType.DMA,
        ],
    )
    return run(ids32, tok_embed, pos_embed)
